# baseline (speedup 1.0000x reference)
import functools
import jax
import jax.numpy as jnp
from jax import lax
from jax.experimental import pallas as pl
from jax.experimental.pallas import tpu as pltpu
from jax.experimental.pallas import tpu_sc as plsc

R = 1024
N = 32768
L = 16
NB = 2048
VREGS = N // L
MIN32 = -2147483648
MASK11 = 0x7FF


@functools.cache
def _build_sort_kernel(rs):
    info = plsc.get_sparse_core_info()
    nw = info.num_cores * info.num_subcores
    assert rs % nw == 0
    rows_per_w = rs // nw
    mesh = plsc.VectorSubcoreMesh(core_axis_name="c", subcore_axis_name="s")

    @functools.partial(
        pl.kernel,
        out_type=[
            jax.ShapeDtypeStruct((rs, N), jnp.float32),
            jax.ShapeDtypeStruct((rs, N), jnp.int32),
        ],
        mesh=mesh,
        compiler_params=pltpu.CompilerParams(needs_layout_passes=False),
        scratch_types=[
            pltpu.VMEM((N,), jnp.float32),
            pltpu.VMEM((N,), jnp.int32),
            pltpu.VMEM((N,), jnp.float32),
            pltpu.VMEM((NB,), jnp.int32),
            pltpu.VMEM((NB,), jnp.int32),
            pltpu.SemaphoreType.DMA,
            pltpu.SemaphoreType.DMA,
            pltpu.SemaphoreType.DMA,
        ],
    )
    def sort_kernel(xf_hbm, vals_hbm, idx_hbm, kbuf, abuf, bbuf, hist, offs,
                    sem_in, sem_ov, sem_oi):
        wid = lax.axis_index("s") * info.num_cores + lax.axis_index("c")
        row0 = wid * rows_per_w
        iota = lax.iota(jnp.int32, L)
        zeros = jnp.zeros((L,), jnp.int32)

        def clear_body(j, _):
            hist[pl.ds(j * L, L)] = zeros
            return 0

        lax.fori_loop(0, NB // L, clear_body, 0)
        pltpu.async_copy(xf_hbm.at[row0], kbuf, sem_in)

        def row_body(r, _):
            row = row0 + r
            pltpu.make_async_copy(xf_hbm.at[row], kbuf, sem_in).wait()

            def conv_body(v, _):
                b = plsc.bitcast(kbuf[pl.ds(v * L, L)], jnp.int32)
                m = jnp.where(b >= 0, b ^ MIN32, ~b)
                kbuf[pl.ds(v * L, L)] = plsc.bitcast(m, jnp.float32)
                dig = m & MASK11
                cnt, lastm = plsc.scan_count(dig)
                plsc.addupdate_scatter(hist, [dig], cnt, mask=lastm)
                return 0

            lax.fori_loop(0, VREGS, conv_body, 0)

            def scan_body(j, carry):
                h = hist[pl.ds(j * L, L)]
                inc = plsc.cumsum(h)
                offs[pl.ds(j * L, L)] = inc - h + carry
                hist[pl.ds(j * L, L)] = zeros
                return carry + jnp.sum(h)

            def permute(i16, m, shift, nshift, dst):
                dig = lax.shift_right_logical(m, shift) & MASK11 if shift else m & MASK11
                cnt, lastm = plsc.scan_count(dig)
                base = plsc.load_gather(offs, [dig])
                payload = plsc.bitcast(i16, jnp.float32) if dst is bbuf else i16
                plsc.store_scatter(dst, [base + cnt], payload)
                plsc.addupdate_scatter(offs, [dig], cnt, mask=lastm)
                if nshift is not None:
                    dig2 = lax.shift_right_logical(m, nshift)
                    if nshift < 22:
                        dig2 = dig2 & MASK11
                    cnt2, lastm2 = plsc.scan_count(dig2)
                    plsc.addupdate_scatter(hist, [dig2], cnt2, mask=lastm2)

            lax.fori_loop(0, NB // L, scan_body, jnp.int32(-1))

            @pl.when(r > 0)
            def _():
                pltpu.make_async_copy(abuf, idx_hbm.at[row - 1], sem_oi).wait()

            def p0_body(v, _):
                m = plsc.bitcast(kbuf[pl.ds(v * L, L)], jnp.int32)
                permute(v * L + iota, m, 0, 11, abuf)
                return 0

            lax.fori_loop(0, VREGS, p0_body, 0)

            lax.fori_loop(0, NB // L, scan_body, jnp.int32(-1))

            @pl.when(r > 0)
            def _():
                pltpu.make_async_copy(bbuf, vals_hbm.at[row - 1], sem_ov).wait()

            def p1_body(v, _):
                i16 = abuf[pl.ds(v * L, L)]
                m = plsc.bitcast(plsc.load_gather(kbuf, [i16]), jnp.int32)
                permute(i16, m, 11, 22, bbuf)
                return 0

            lax.fori_loop(0, VREGS, p1_body, 0)

            lax.fori_loop(0, NB // L, scan_body, jnp.int32(-1))

            def p2_body(v, _):
                i16 = plsc.bitcast(bbuf[pl.ds(v * L, L)], jnp.int32)
                m = plsc.bitcast(plsc.load_gather(kbuf, [i16]), jnp.int32)
                permute(i16, m, 22, None, abuf)
                return 0

            lax.fori_loop(0, VREGS, p2_body, 0)

            def fin_body(v, _):
                i16 = abuf[pl.ds(v * L, L)]
                m = plsc.bitcast(plsc.load_gather(kbuf, [i16]), jnp.int32)
                inv = jnp.where(m < 0, m ^ MIN32, ~m)
                bbuf[pl.ds(v * L, L)] = plsc.bitcast(inv, jnp.float32)
                return 0

            lax.fori_loop(0, VREGS, fin_body, 0)

            pltpu.async_copy(abuf, idx_hbm.at[row], sem_oi)
            pltpu.async_copy(bbuf, vals_hbm.at[row], sem_ov)

            @pl.when(r < rows_per_w - 1)
            def _():
                pltpu.async_copy(xf_hbm.at[row + 1], kbuf, sem_in)

            return 0

        lax.fori_loop(0, rows_per_w, row_body, 0)
        last = row0 + rows_per_w - 1
        pltpu.make_async_copy(abuf, idx_hbm.at[last], sem_oi).wait()
        pltpu.make_async_copy(bbuf, vals_hbm.at[last], sem_ov).wait()

    return sort_kernel


TC_RB = 8


def _ce(lo_v, hi_v, lo_i, hi_i, desc):
    gt = (lo_v > hi_v) | ((lo_v == hi_v) & (lo_i > hi_i))
    sw = gt ^ desc
    return (jnp.where(sw, hi_v, lo_v), jnp.where(sw, lo_v, hi_v),
            jnp.where(sw, hi_i, lo_i), jnp.where(sw, lo_i, hi_i))


def _step_A(v, ix, s, t):
    S = 1 << (t - 8)
    RB = v.shape[0]
    M = 128 // (2 * S)
    vr = v.reshape(RB, M, 2, S, 256)
    ir = ix.reshape(RB, M, 2, S, 256)
    mi = lax.broadcasted_iota(jnp.int32, (1, M, 1, 1), 1)
    desc = ((mi >> (s - t - 1)) & 1) == 1
    nlo_v, nhi_v, nlo_i, nhi_i = _ce(vr[:, :, 0], vr[:, :, 1],
                                     ir[:, :, 0], ir[:, :, 1], desc)
    v2 = jnp.stack([nlo_v, nhi_v], axis=2).reshape(RB, 128, 256)
    i2 = jnp.stack([nlo_i, nhi_i], axis=2).reshape(RB, 128, 256)
    return v2, i2


def _step_B(v, ix, s, t):
    S = 1 << t
    RB = v.shape[0]
    M = 256 // (2 * S)
    vr = v.reshape(RB, M, 2, S, 128)
    ir = ix.reshape(RB, M, 2, S, 128)
    if s <= 7:
        mi = lax.broadcasted_iota(jnp.int32, (1, M, 1, 1), 1)
        desc = ((mi >> (s - t - 1)) & 1) == 1
    else:
        li = lax.broadcasted_iota(jnp.int32, (1, 1, 1, 128), 3)
        desc = ((li >> (s - 8)) & 1) == 1
    nlo_v, nhi_v, nlo_i, nhi_i = _ce(vr[:, :, 0], vr[:, :, 1],
                                     ir[:, :, 0], ir[:, :, 1], desc)
    v2 = jnp.stack([nlo_v, nhi_v], axis=2).reshape(RB, 256, 128)
    i2 = jnp.stack([nlo_i, nhi_i], axis=2).reshape(RB, 256, 128)
    return v2, i2


def _tc_sort_body(x_ref, v_ref, i_ref):
    RB = x_ref.shape[0]
    v = x_ref[...].reshape(RB, 128, 256)
    ix = lax.broadcasted_iota(jnp.int32, (RB, N), 1).reshape(RB, 128, 256)
    v = jnp.swapaxes(v, 1, 2)
    ix = jnp.swapaxes(ix, 1, 2)
    for s in range(1, 16):
        if s >= 9:
            v = jnp.swapaxes(v, 1, 2)
            ix = jnp.swapaxes(ix, 1, 2)
            for t in range(s - 1, 7, -1):
                v, ix = _step_A(v, ix, s, t)
            v = jnp.swapaxes(v, 1, 2)
            ix = jnp.swapaxes(ix, 1, 2)
        for t in range(min(s - 1, 7), -1, -1):
            v, ix = _step_B(v, ix, s, t)
    v = jnp.swapaxes(v, 1, 2)
    ix = jnp.swapaxes(ix, 1, 2)
    v_ref[...] = v.reshape(RB, N)
    i_ref[...] = ix.reshape(RB, N)


def _tc_sort(x):
    rows = x.shape[0]
    return pl.pallas_call(
        _tc_sort_body,
        grid=(rows // TC_RB,),
        in_specs=[pl.BlockSpec((TC_RB, N), lambda i: (i, 0))],
        out_specs=[pl.BlockSpec((TC_RB, N), lambda i: (i, 0)),
                   pl.BlockSpec((TC_RB, N), lambda i: (i, 0))],
        out_shape=[jax.ShapeDtypeStruct((rows, N), jnp.float32),
                   jax.ShapeDtypeStruct((rows, N), jnp.int32)],
    )(x)


R_SC = 384


@jax.jit
def kernel(input_tensors):
    x = input_tensors
    v0, i0 = _build_sort_kernel(R_SC)(x[:R_SC])
    v1, i1 = _tc_sort(x[R_SC:])
    return (jnp.concatenate([v0, v1]), jnp.concatenate([i0, i1]))


# speedup vs baseline: 2.7437x; 2.7437x over previous
import functools
import jax
import jax.numpy as jnp
from jax import lax
from jax.experimental import pallas as pl
from jax.experimental.pallas import tpu as pltpu
from jax.experimental.pallas import tpu_sc as plsc

R = 1024
N = 32768
L = 16
NB = 2048
VREGS = N // L
MIN32 = -2147483648
MASK11 = 0x7FF


@functools.cache
def _build_sort_kernel(rs):
    info = plsc.get_sparse_core_info()
    nw = info.num_cores * info.num_subcores
    assert rs % nw == 0
    rows_per_w = rs // nw
    mesh = plsc.VectorSubcoreMesh(core_axis_name="c", subcore_axis_name="s")

    @functools.partial(
        pl.kernel,
        out_type=[
            jax.ShapeDtypeStruct((rs, N), jnp.float32),
            jax.ShapeDtypeStruct((rs, N), jnp.int32),
        ],
        mesh=mesh,
        compiler_params=pltpu.CompilerParams(needs_layout_passes=False),
        scratch_types=[
            pltpu.VMEM((N,), jnp.float32),
            pltpu.VMEM((N,), jnp.int32),
            pltpu.VMEM((N,), jnp.float32),
            pltpu.VMEM((NB,), jnp.int32),
            pltpu.VMEM((NB,), jnp.int32),
            pltpu.SemaphoreType.DMA,
            pltpu.SemaphoreType.DMA,
            pltpu.SemaphoreType.DMA,
        ],
    )
    def sort_kernel(xf_hbm, vals_hbm, idx_hbm, kbuf, abuf, bbuf, hist, offs,
                    sem_in, sem_ov, sem_oi):
        wid = lax.axis_index("s") * info.num_cores + lax.axis_index("c")
        row0 = wid * rows_per_w
        iota = lax.iota(jnp.int32, L)
        zeros = jnp.zeros((L,), jnp.int32)

        def clear_body(j, _):
            hist[pl.ds(j * L, L)] = zeros
            return 0

        lax.fori_loop(0, NB // L, clear_body, 0)
        pltpu.async_copy(xf_hbm.at[row0], kbuf, sem_in)

        def row_body(r, _):
            row = row0 + r
            pltpu.make_async_copy(xf_hbm.at[row], kbuf, sem_in).wait()

            def conv_body(v, _):
                b = plsc.bitcast(kbuf[pl.ds(v * L, L)], jnp.int32)
                m = jnp.where(b >= 0, b ^ MIN32, ~b)
                kbuf[pl.ds(v * L, L)] = plsc.bitcast(m, jnp.float32)
                dig = m & MASK11
                cnt, lastm = plsc.scan_count(dig)
                plsc.addupdate_scatter(hist, [dig], cnt, mask=lastm)
                return 0

            lax.fori_loop(0, VREGS, conv_body, 0)

            def scan_body(j, carry):
                h = hist[pl.ds(j * L, L)]
                inc = plsc.cumsum(h)
                offs[pl.ds(j * L, L)] = inc - h + carry
                hist[pl.ds(j * L, L)] = zeros
                return carry + jnp.sum(h)

            def permute(i16, m, shift, nshift, dst):
                dig = lax.shift_right_logical(m, shift) & MASK11 if shift else m & MASK11
                cnt, lastm = plsc.scan_count(dig)
                base = plsc.load_gather(offs, [dig])
                payload = plsc.bitcast(i16, jnp.float32) if dst is bbuf else i16
                plsc.store_scatter(dst, [base + cnt], payload)
                plsc.addupdate_scatter(offs, [dig], cnt, mask=lastm)
                if nshift is not None:
                    dig2 = lax.shift_right_logical(m, nshift)
                    if nshift < 22:
                        dig2 = dig2 & MASK11
                    cnt2, lastm2 = plsc.scan_count(dig2)
                    plsc.addupdate_scatter(hist, [dig2], cnt2, mask=lastm2)

            lax.fori_loop(0, NB // L, scan_body, jnp.int32(-1))

            @pl.when(r > 0)
            def _():
                pltpu.make_async_copy(abuf, idx_hbm.at[row - 1], sem_oi).wait()

            def p0_body(v, _):
                m = plsc.bitcast(kbuf[pl.ds(v * L, L)], jnp.int32)
                permute(v * L + iota, m, 0, 11, abuf)
                return 0

            lax.fori_loop(0, VREGS, p0_body, 0)

            lax.fori_loop(0, NB // L, scan_body, jnp.int32(-1))

            @pl.when(r > 0)
            def _():
                pltpu.make_async_copy(bbuf, vals_hbm.at[row - 1], sem_ov).wait()

            def p1_body(v, _):
                i16 = abuf[pl.ds(v * L, L)]
                m = plsc.bitcast(plsc.load_gather(kbuf, [i16]), jnp.int32)
                permute(i16, m, 11, 22, bbuf)
                return 0

            lax.fori_loop(0, VREGS, p1_body, 0)

            lax.fori_loop(0, NB // L, scan_body, jnp.int32(-1))

            def p2_body(v, _):
                i16 = plsc.bitcast(bbuf[pl.ds(v * L, L)], jnp.int32)
                m = plsc.bitcast(plsc.load_gather(kbuf, [i16]), jnp.int32)
                permute(i16, m, 22, None, abuf)
                return 0

            lax.fori_loop(0, VREGS, p2_body, 0)

            def fin_body(v, _):
                i16 = abuf[pl.ds(v * L, L)]
                m = plsc.bitcast(plsc.load_gather(kbuf, [i16]), jnp.int32)
                inv = jnp.where(m < 0, m ^ MIN32, ~m)
                bbuf[pl.ds(v * L, L)] = plsc.bitcast(inv, jnp.float32)
                return 0

            lax.fori_loop(0, VREGS, fin_body, 0)

            pltpu.async_copy(abuf, idx_hbm.at[row], sem_oi)
            pltpu.async_copy(bbuf, vals_hbm.at[row], sem_ov)

            @pl.when(r < rows_per_w - 1)
            def _():
                pltpu.async_copy(xf_hbm.at[row + 1], kbuf, sem_in)

            return 0

        lax.fori_loop(0, rows_per_w, row_body, 0)
        last = row0 + rows_per_w - 1
        pltpu.make_async_copy(abuf, idx_hbm.at[last], sem_oi).wait()
        pltpu.make_async_copy(bbuf, vals_hbm.at[last], sem_ov).wait()

    return sort_kernel


TC_RB = 8


def _ce(lo_v, hi_v, lo_i, hi_i, desc):
    gt = (lo_v > hi_v) | ((lo_v == hi_v) & (lo_i > hi_i))
    sw = gt ^ desc
    return (jnp.where(sw, hi_v, lo_v), jnp.where(sw, lo_v, hi_v),
            jnp.where(sw, hi_i, lo_i), jnp.where(sw, lo_i, hi_i))


def _step_A(v, ix, s, t):
    S = 1 << (t - 8)
    RB = v.shape[0]
    M = 128 // (2 * S)
    vr = v.reshape(RB, M, 2, S, 256)
    ir = ix.reshape(RB, M, 2, S, 256)
    mi = lax.broadcasted_iota(jnp.int32, (1, M, 1, 1), 1)
    desc = ((mi >> (s - t - 1)) & 1) == 1
    nlo_v, nhi_v, nlo_i, nhi_i = _ce(vr[:, :, 0], vr[:, :, 1],
                                     ir[:, :, 0], ir[:, :, 1], desc)
    v2 = jnp.stack([nlo_v, nhi_v], axis=2).reshape(RB, 128, 256)
    i2 = jnp.stack([nlo_i, nhi_i], axis=2).reshape(RB, 128, 256)
    return v2, i2


def _step_B(v, ix, s, t):
    S = 1 << t
    RB = v.shape[0]
    M = 256 // (2 * S)
    vr = v.reshape(RB, M, 2, S, 128)
    ir = ix.reshape(RB, M, 2, S, 128)
    if s <= 7:
        mi = lax.broadcasted_iota(jnp.int32, (1, M, 1, 1), 1)
        desc = ((mi >> (s - t - 1)) & 1) == 1
    else:
        li = lax.broadcasted_iota(jnp.int32, (1, 1, 1, 128), 3)
        desc = ((li >> (s - 8)) & 1) == 1
    nlo_v, nhi_v, nlo_i, nhi_i = _ce(vr[:, :, 0], vr[:, :, 1],
                                     ir[:, :, 0], ir[:, :, 1], desc)
    v2 = jnp.stack([nlo_v, nhi_v], axis=2).reshape(RB, 256, 128)
    i2 = jnp.stack([nlo_i, nhi_i], axis=2).reshape(RB, 256, 128)
    return v2, i2


def _tc_sort_body(x_ref, v_ref, i_ref):
    RB = x_ref.shape[0]
    v = x_ref[...].reshape(RB, 128, 256)
    ix = lax.broadcasted_iota(jnp.int32, (RB, N), 1).reshape(RB, 128, 256)
    v = jnp.swapaxes(v, 1, 2)
    ix = jnp.swapaxes(ix, 1, 2)
    for s in range(1, 16):
        if s >= 9:
            v = jnp.swapaxes(v, 1, 2)
            ix = jnp.swapaxes(ix, 1, 2)
            for t in range(s - 1, 7, -1):
                v, ix = _step_A(v, ix, s, t)
            v = jnp.swapaxes(v, 1, 2)
            ix = jnp.swapaxes(ix, 1, 2)
        for t in range(min(s - 1, 7), -1, -1):
            v, ix = _step_B(v, ix, s, t)
    v = jnp.swapaxes(v, 1, 2)
    ix = jnp.swapaxes(ix, 1, 2)
    v_ref[...] = v.reshape(RB, N)
    i_ref[...] = ix.reshape(RB, N)


def _tc_sort(x):
    rows = x.shape[0]
    return pl.pallas_call(
        _tc_sort_body,
        grid=(rows // TC_RB,),
        in_specs=[pl.BlockSpec((TC_RB, N), lambda i: (i, 0))],
        out_specs=[pl.BlockSpec((TC_RB, N), lambda i: (i, 0)),
                   pl.BlockSpec((TC_RB, N), lambda i: (i, 0))],
        out_shape=[jax.ShapeDtypeStruct((rows, N), jnp.float32),
                   jax.ShapeDtypeStruct((rows, N), jnp.int32)],
    )(x)


R_SC = 800


@jax.jit
def kernel(input_tensors):
    x = input_tensors
    v0, i0 = _build_sort_kernel(R_SC)(x[:R_SC])
    v1, i1 = _tc_sort(x[R_SC:])
    return (jnp.concatenate([v0, v1]), jnp.concatenate([i0, i1]))


# speedup vs baseline: 2.7485x; 1.0017x over previous
import functools
import jax
import jax.numpy as jnp
from jax import lax
from jax.experimental import pallas as pl
from jax.experimental.pallas import tpu as pltpu
from jax.experimental.pallas import tpu_sc as plsc

R = 1024
N = 32768
L = 16
NB = 2048
VREGS = N // L
MIN32 = -2147483648
MASK11 = 0x7FF


@functools.cache
def _build_sort_kernel(rs):
    info = plsc.get_sparse_core_info()
    nw = info.num_cores * info.num_subcores
    assert rs % nw == 0
    rows_per_w = rs // nw
    mesh = plsc.VectorSubcoreMesh(core_axis_name="c", subcore_axis_name="s")

    @functools.partial(
        pl.kernel,
        out_type=[
            jax.ShapeDtypeStruct((rs, N), jnp.float32),
            jax.ShapeDtypeStruct((rs, N), jnp.int32),
        ],
        mesh=mesh,
        compiler_params=pltpu.CompilerParams(needs_layout_passes=False),
        scratch_types=[
            pltpu.VMEM((N,), jnp.float32),
            pltpu.VMEM((N,), jnp.int32),
            pltpu.VMEM((N,), jnp.float32),
            pltpu.VMEM((NB,), jnp.int32),
            pltpu.VMEM((NB,), jnp.int32),
            pltpu.SemaphoreType.DMA,
            pltpu.SemaphoreType.DMA,
            pltpu.SemaphoreType.DMA,
        ],
    )
    def sort_kernel(xf_hbm, vals_hbm, idx_hbm, kbuf, abuf, bbuf, hist, offs,
                    sem_in, sem_ov, sem_oi):
        wid = lax.axis_index("s") * info.num_cores + lax.axis_index("c")
        row0 = wid * rows_per_w
        iota = lax.iota(jnp.int32, L)
        zeros = jnp.zeros((L,), jnp.int32)

        def clear_body(j, _):
            hist[pl.ds(j * L, L)] = zeros
            return 0

        lax.fori_loop(0, NB // L, clear_body, 0)
        pltpu.async_copy(xf_hbm.at[row0], kbuf, sem_in)

        def row_body(r, _):
            row = row0 + r
            pltpu.make_async_copy(xf_hbm.at[row], kbuf, sem_in).wait()

            def conv_body(v, _):
                b = plsc.bitcast(kbuf[pl.ds(v * L, L)], jnp.int32)
                m = jnp.where(b >= 0, b ^ MIN32, ~b)
                kbuf[pl.ds(v * L, L)] = plsc.bitcast(m, jnp.float32)
                dig = m & MASK11
                cnt, lastm = plsc.scan_count(dig)
                plsc.addupdate_scatter(hist, [dig], cnt, mask=lastm)
                return 0

            lax.fori_loop(0, VREGS, conv_body, 0)

            def scan_body(j, carry):
                h = hist[pl.ds(j * L, L)]
                inc = plsc.cumsum(h)
                offs[pl.ds(j * L, L)] = inc - h + carry
                hist[pl.ds(j * L, L)] = zeros
                return carry + jnp.sum(h)

            def permute(i16, m, shift, nshift, dst):
                dig = lax.shift_right_logical(m, shift) & MASK11 if shift else m & MASK11
                cnt, lastm = plsc.scan_count(dig)
                base = plsc.load_gather(offs, [dig])
                payload = plsc.bitcast(i16, jnp.float32) if dst is bbuf else i16
                plsc.store_scatter(dst, [base + cnt], payload)
                plsc.addupdate_scatter(offs, [dig], cnt, mask=lastm)
                if nshift is not None:
                    dig2 = lax.shift_right_logical(m, nshift)
                    if nshift < 22:
                        dig2 = dig2 & MASK11
                    cnt2, lastm2 = plsc.scan_count(dig2)
                    plsc.addupdate_scatter(hist, [dig2], cnt2, mask=lastm2)

            lax.fori_loop(0, NB // L, scan_body, jnp.int32(-1))

            @pl.when(r > 0)
            def _():
                pltpu.make_async_copy(abuf, idx_hbm.at[row - 1], sem_oi).wait()

            def p0_body(v, _):
                m = plsc.bitcast(kbuf[pl.ds(v * L, L)], jnp.int32)
                permute(v * L + iota, m, 0, 11, abuf)
                return 0

            lax.fori_loop(0, VREGS, p0_body, 0)

            lax.fori_loop(0, NB // L, scan_body, jnp.int32(-1))

            @pl.when(r > 0)
            def _():
                pltpu.make_async_copy(bbuf, vals_hbm.at[row - 1], sem_ov).wait()

            def p1_body(v, _):
                i16 = abuf[pl.ds(v * L, L)]
                m = plsc.bitcast(plsc.load_gather(kbuf, [i16]), jnp.int32)
                permute(i16, m, 11, 22, bbuf)
                return 0

            lax.fori_loop(0, VREGS, p1_body, 0)

            lax.fori_loop(0, NB // L, scan_body, jnp.int32(-1))

            def p2_body(v, _):
                i16 = plsc.bitcast(bbuf[pl.ds(v * L, L)], jnp.int32)
                m = plsc.bitcast(plsc.load_gather(kbuf, [i16]), jnp.int32)
                permute(i16, m, 22, None, abuf)
                return 0

            lax.fori_loop(0, VREGS, p2_body, 0)

            def fin_body(v, _):
                i16 = abuf[pl.ds(v * L, L)]
                m = plsc.bitcast(plsc.load_gather(kbuf, [i16]), jnp.int32)
                inv = jnp.where(m < 0, m ^ MIN32, ~m)
                bbuf[pl.ds(v * L, L)] = plsc.bitcast(inv, jnp.float32)
                return 0

            lax.fori_loop(0, VREGS, fin_body, 0)

            pltpu.async_copy(abuf, idx_hbm.at[row], sem_oi)
            pltpu.async_copy(bbuf, vals_hbm.at[row], sem_ov)

            @pl.when(r < rows_per_w - 1)
            def _():
                pltpu.async_copy(xf_hbm.at[row + 1], kbuf, sem_in)

            return 0

        lax.fori_loop(0, rows_per_w, row_body, 0)
        last = row0 + rows_per_w - 1
        pltpu.make_async_copy(abuf, idx_hbm.at[last], sem_oi).wait()
        pltpu.make_async_copy(bbuf, vals_hbm.at[last], sem_ov).wait()

    return sort_kernel


TC_RB = 16


def _ce(lo_v, hi_v, lo_i, hi_i, desc):
    gt = (lo_v > hi_v) | ((lo_v == hi_v) & (lo_i > hi_i))
    sw = gt ^ desc
    return (jnp.where(sw, hi_v, lo_v), jnp.where(sw, lo_v, hi_v),
            jnp.where(sw, hi_i, lo_i), jnp.where(sw, lo_i, hi_i))


def _step_A(v, ix, s, t):
    S = 1 << (t - 8)
    RB = v.shape[0]
    M = 128 // (2 * S)
    vr = v.reshape(RB, M, 2, S, 256)
    ir = ix.reshape(RB, M, 2, S, 256)
    mi = lax.broadcasted_iota(jnp.int32, (1, M, 1, 1), 1)
    desc = ((mi >> (s - t - 1)) & 1) == 1
    nlo_v, nhi_v, nlo_i, nhi_i = _ce(vr[:, :, 0], vr[:, :, 1],
                                     ir[:, :, 0], ir[:, :, 1], desc)
    v2 = jnp.stack([nlo_v, nhi_v], axis=2).reshape(RB, 128, 256)
    i2 = jnp.stack([nlo_i, nhi_i], axis=2).reshape(RB, 128, 256)
    return v2, i2


def _step_B(v, ix, s, t):
    S = 1 << t
    RB = v.shape[0]
    M = 256 // (2 * S)
    vr = v.reshape(RB, M, 2, S, 128)
    ir = ix.reshape(RB, M, 2, S, 128)
    if s <= 7:
        mi = lax.broadcasted_iota(jnp.int32, (1, M, 1, 1), 1)
        desc = ((mi >> (s - t - 1)) & 1) == 1
    else:
        li = lax.broadcasted_iota(jnp.int32, (1, 1, 1, 128), 3)
        desc = ((li >> (s - 8)) & 1) == 1
    nlo_v, nhi_v, nlo_i, nhi_i = _ce(vr[:, :, 0], vr[:, :, 1],
                                     ir[:, :, 0], ir[:, :, 1], desc)
    v2 = jnp.stack([nlo_v, nhi_v], axis=2).reshape(RB, 256, 128)
    i2 = jnp.stack([nlo_i, nhi_i], axis=2).reshape(RB, 256, 128)
    return v2, i2


def _tc_sort_body(x_ref, v_ref, i_ref):
    RB = x_ref.shape[0]
    v = x_ref[...].reshape(RB, 128, 256)
    ix = lax.broadcasted_iota(jnp.int32, (RB, N), 1).reshape(RB, 128, 256)
    v = jnp.swapaxes(v, 1, 2)
    ix = jnp.swapaxes(ix, 1, 2)
    for s in range(1, 16):
        if s >= 9:
            v = jnp.swapaxes(v, 1, 2)
            ix = jnp.swapaxes(ix, 1, 2)
            for t in range(s - 1, 7, -1):
                v, ix = _step_A(v, ix, s, t)
            v = jnp.swapaxes(v, 1, 2)
            ix = jnp.swapaxes(ix, 1, 2)
        for t in range(min(s - 1, 7), -1, -1):
            v, ix = _step_B(v, ix, s, t)
    v = jnp.swapaxes(v, 1, 2)
    ix = jnp.swapaxes(ix, 1, 2)
    v_ref[...] = v.reshape(RB, N)
    i_ref[...] = ix.reshape(RB, N)


def _tc_sort(x):
    rows = x.shape[0]
    return pl.pallas_call(
        _tc_sort_body,
        grid=(rows // TC_RB,),
        in_specs=[pl.BlockSpec((TC_RB, N), lambda i: (i, 0))],
        out_specs=[pl.BlockSpec((TC_RB, N), lambda i: (i, 0)),
                   pl.BlockSpec((TC_RB, N), lambda i: (i, 0))],
        out_shape=[jax.ShapeDtypeStruct((rows, N), jnp.float32),
                   jax.ShapeDtypeStruct((rows, N), jnp.int32)],
    )(x)


R_SC = 800


@jax.jit
def kernel(input_tensors):
    x = input_tensors
    v0, i0 = _build_sort_kernel(R_SC)(x[:R_SC])
    v1, i1 = _tc_sort(x[R_SC:])
    return (jnp.concatenate([v0, v1]), jnp.concatenate([i0, i1]))
